# baseline (speedup 1.0000x reference)
import functools

import jax
import jax.numpy as jnp
from jax.experimental import pallas as pl
from jax.experimental.pallas import tpu as pltpu

EPS = 1e-5
PAD = 128


def _round_up(x, m):
    return (x + m - 1) // m * m


def _pick_pos_tile(P, Wo):
    tp = Wo
    for cand in range(1, P // Wo + 1):
        if P % (cand * Wo) == 0 and cand * Wo <= 512:
            tp = cand * Wo
    return tp


def _make_fused_kernel(KH, KW, Ho, Wo, TP, Cin, CB):
    P = Ho * Wo
    n_chunks = P // TP
    Q = PAD + P + PAD

    def _body(x_ref, w_ref, g_ref, bt_ref, o_ref, xq, lhs, y_scr):
        xq[:, :PAD] = jnp.zeros((Cin, PAD), jnp.bfloat16)
        xq[:, PAD + P:] = jnp.zeros((Cin, Q - PAD - P), jnp.bfloat16)
        xq[:, PAD:PAD + P] = x_ref[0]
        w_mat = w_ref[...]

        lane_w = jax.lax.broadcasted_iota(jnp.int32, (Cin, TP), 1) % Wo
        mask_l = lane_w != 0
        mask_r = lane_w != (Wo - 1)
        zero = jnp.zeros((Cin, TP), jnp.bfloat16)

        s_acc = jnp.zeros((CB, TP), jnp.float32)
        ss_acc = jnp.zeros((CB, TP), jnp.float32)
        for c in range(n_chunks):
            p0 = c * TP
            buf = lhs.at[c % 2]
            for i in range(KH):
                for j in range(KW):
                    start = PAD + p0 + (i - (KH // 2)) * Wo + (j - (KW // 2))
                    tap = xq[:, pl.ds(start, TP)]
                    if j == 0:
                        tap = jnp.where(mask_l, tap, zero)
                    elif j == KW - 1:
                        tap = jnp.where(mask_r, tap, zero)
                    r0 = (i * KW + j) * Cin
                    buf[r0:r0 + Cin, :] = tap
            y = jnp.dot(w_mat, buf[...],
                        preferred_element_type=jnp.float32)
            y_scr[:, p0:p0 + TP] = y
            s_acc = s_acc + y
            ss_acc = ss_acc + y * y

        inv_p = 1.0 / float(P)
        s = jnp.sum(s_acc, axis=1, keepdims=True)
        ss = jnp.sum(ss_acc, axis=1, keepdims=True)
        mean = s * inv_p
        var = jnp.maximum(ss * inv_p - mean * mean, 0.0)
        scale = jax.lax.rsqrt(var + EPS) * g_ref[...]
        shift = bt_ref[...] - mean * scale

        o_ref[0] = jnp.maximum(y_scr[...] * scale + shift,
                               0.0).astype(jnp.bfloat16)

    return _body


@functools.partial(jax.jit, static_argnames=("stride", "padding"))
def _fused_conv_in_relu(x_nchw, w_oihw, gamma, beta, *, stride=1, padding=0):
    N, Cin, H, W = x_nchw.shape
    Cout, Cin_w, KH, KW = w_oihw.shape
    assert Cin == Cin_w and stride == 1
    assert padding == KH // 2 == KW // 2, "same-size conv expected"
    assert PAD >= padding * W + padding

    Ho, Wo = H, W
    P = Ho * Wo
    K = KH * KW * Cin
    CB = 128
    Cp = _round_up(Cout, CB)
    assert Cp == CB, "single 128-channel block expected"

    x_flat = x_nchw.reshape(N, Cin, P).astype(jnp.bfloat16)
    w = jnp.transpose(w_oihw, (0, 2, 3, 1)).reshape(Cout, K)
    w = jnp.pad(w, ((0, Cp - Cout), (0, 0))).astype(jnp.bfloat16)
    gp = jnp.pad(gamma, (0, Cp - Cout)).reshape(Cp, 1)
    btp = jnp.pad(beta, (0, Cp - Cout)).reshape(Cp, 1)

    TP = _pick_pos_tile(P, Wo)
    body = _make_fused_kernel(KH, KW, Ho, Wo, TP, Cin, CB)

    out = pl.pallas_call(
        body,
        out_shape=jax.ShapeDtypeStruct((N, Cp, P), jnp.bfloat16),
        grid=(N,),
        in_specs=[
            pl.BlockSpec((1, Cin, P), lambda n: (n, 0, 0)),
            pl.BlockSpec((CB, K), lambda n: (0, 0)),
            pl.BlockSpec((CB, 1), lambda n: (0, 0)),
            pl.BlockSpec((CB, 1), lambda n: (0, 0)),
        ],
        out_specs=pl.BlockSpec((1, CB, P), lambda n: (n, 0, 0)),
        scratch_shapes=[
            pltpu.VMEM((Cin, PAD + P + PAD), jnp.bfloat16),
            pltpu.VMEM((2, K, TP), jnp.bfloat16),
            pltpu.VMEM((CB, P), jnp.float32),
        ],
        compiler_params=pltpu.CompilerParams(
            dimension_semantics=("parallel",)),
    )(x_flat, w, gp, btp)

    return out[:, :Cout, :].reshape(N, Cout, Ho, Wo).astype(jnp.float32)


def kernel(x, w, b, gamma, beta):
    del b
    return _fused_conv_in_relu(x, w, gamma, beta, stride=1, padding=1)


# speedup vs baseline: 1.2038x; 1.2038x over previous
import functools

import jax
import jax.numpy as jnp
from jax.experimental import pallas as pl
from jax.experimental.pallas import tpu as pltpu

EPS = 1e-5
PAD = 128
TP = 512


def _round_up(x, m):
    return (x + m - 1) // m * m


def _make_fused_kernel(KH, KW, Ho, Wo, Cin, CB):
    P = Ho * Wo
    Pr = _round_up(P, TP)
    n_chunks = Pr // TP
    Q = PAD + Pr + PAD

    def _body(x_ref, w_ref, g_ref, bt_ref, o_ref, xq, lhs, y_scr):
        xq[:, :PAD] = jnp.zeros((Cin, PAD), jnp.bfloat16)
        xq[:, PAD + P:] = jnp.zeros((Cin, Q - PAD - P), jnp.bfloat16)
        xq[:, PAD:PAD + P] = x_ref[0]
        w_mat = w_ref[...]

        lane_w = jax.lax.broadcasted_iota(jnp.int32, (Cin, Pr), 1) % Wo
        mask_l = lane_w != 0
        mask_r = lane_w != (Wo - 1)
        zero = jnp.zeros((Cin, Pr), jnp.bfloat16)

        for i in range(KH):
            for j in range(KW):
                start = PAD + (i - (KH // 2)) * Wo + (j - (KW // 2))
                tap = xq[:, pl.ds(start, Pr)]
                if j == 0:
                    tap = jnp.where(mask_l, tap, zero)
                elif j == KW - 1:
                    tap = jnp.where(mask_r, tap, zero)
                r0 = (i * KW + j) * Cin
                lhs[r0:r0 + Cin, :] = tap

        s_acc = jnp.zeros((CB, TP), jnp.float32)
        ss_acc = jnp.zeros((CB, TP), jnp.float32)
        pvalid = jax.lax.broadcasted_iota(jnp.int32, (CB, TP), 1)
        for c in range(n_chunks):
            y = jnp.dot(w_mat, lhs[:, c * TP:(c + 1) * TP],
                        preferred_element_type=jnp.float32)
            y_scr[:, c * TP:(c + 1) * TP] = y
            if (c + 1) * TP > P:
                y = jnp.where(pvalid < (P - c * TP), y, 0.0)
            s_acc = s_acc + y
            ss_acc = ss_acc + y * y

        inv_p = 1.0 / float(P)
        s = jnp.sum(s_acc, axis=1, keepdims=True)
        ss = jnp.sum(ss_acc, axis=1, keepdims=True)
        mean = s * inv_p
        var = jnp.maximum(ss * inv_p - mean * mean, 0.0)
        scale = jax.lax.rsqrt(var + EPS) * g_ref[...]
        shift = bt_ref[...] - mean * scale

        o_ref[0] = jnp.maximum(y_scr[:, :P] * scale + shift, 0.0)

    return _body


@functools.partial(jax.jit, static_argnames=("stride", "padding"))
def _fused_conv_in_relu(x_nchw, w_oihw, gamma, beta, *, stride=1, padding=0):
    N, Cin, H, W = x_nchw.shape
    Cout, Cin_w, KH, KW = w_oihw.shape
    assert Cin == Cin_w and stride == 1
    assert padding == KH // 2 == KW // 2, "same-size conv expected"
    assert PAD >= padding * W + padding

    Ho, Wo = H, W
    P = Ho * Wo
    K = KH * KW * Cin
    CB = 128
    Cp = _round_up(Cout, CB)
    assert Cp == CB, "single 128-channel block expected"
    Pr = _round_up(P, TP)

    x_flat = x_nchw.reshape(N, Cin, P).astype(jnp.bfloat16)
    w = jnp.transpose(w_oihw, (0, 2, 3, 1)).reshape(Cout, K)
    w = jnp.pad(w, ((0, Cp - Cout), (0, 0))).astype(jnp.bfloat16)
    gp = jnp.pad(gamma, (0, Cp - Cout)).reshape(Cp, 1)
    btp = jnp.pad(beta, (0, Cp - Cout)).reshape(Cp, 1)

    body = _make_fused_kernel(KH, KW, Ho, Wo, Cin, CB)

    out = pl.pallas_call(
        body,
        out_shape=jax.ShapeDtypeStruct((N, Cp, P), jnp.float32),
        grid=(N,),
        in_specs=[
            pl.BlockSpec((1, Cin, P), lambda n: (n, 0, 0)),
            pl.BlockSpec((CB, K), lambda n: (0, 0)),
            pl.BlockSpec((CB, 1), lambda n: (0, 0)),
            pl.BlockSpec((CB, 1), lambda n: (0, 0)),
        ],
        out_specs=pl.BlockSpec((1, CB, P), lambda n: (n, 0, 0)),
        scratch_shapes=[
            pltpu.VMEM((Cin, PAD + Pr + PAD), jnp.bfloat16),
            pltpu.VMEM((K, Pr), jnp.bfloat16),
            pltpu.VMEM((CB, Pr), jnp.float32),
        ],
        compiler_params=pltpu.CompilerParams(
            dimension_semantics=("parallel",)),
    )(x_flat, w, gp, btp)

    return out[:, :Cout, :].reshape(N, Cout, Ho, Wo)


def kernel(x, w, b, gamma, beta):
    del b
    return _fused_conv_in_relu(x, w, gamma, beta, stride=1, padding=1)


# speedup vs baseline: 1.2068x; 1.0024x over previous
import functools

import jax
import jax.numpy as jnp
from jax.experimental import pallas as pl
from jax.experimental.pallas import tpu as pltpu

EPS = 1e-5
PAD = 128
TP = 512


def _round_up(x, m):
    return (x + m - 1) // m * m


def _make_fused_kernel(KH, KW, Ho, Wo, Cin, CB):
    P = Ho * Wo
    Pr = _round_up(P, TP)
    n_chunks = Pr // TP
    Q = PAD + Pr + PAD

    def _body(x_ref, w_ref, g_ref, bt_ref, o_ref, xq, lhs, y_scr):
        xq[:, :PAD] = jnp.zeros((Cin, PAD), jnp.bfloat16)
        xq[:, PAD + P:] = jnp.zeros((Cin, Q - PAD - P), jnp.bfloat16)
        xq[:, PAD:PAD + P] = x_ref[0]
        w_mat = w_ref[...]

        lane_w = jax.lax.broadcasted_iota(jnp.int32, (Cin, Pr), 1) % Wo
        mask_l = lane_w != 0
        mask_r = lane_w != (Wo - 1)
        zero = jnp.zeros((Cin, Pr), jnp.bfloat16)

        for i in range(KH):
            for j in range(KW):
                start = PAD + (i - (KH // 2)) * Wo + (j - (KW // 2))
                tap = xq[:, pl.ds(start, Pr)]
                if j == 0:
                    tap = jnp.where(mask_l, tap, zero)
                elif j == KW - 1:
                    tap = jnp.where(mask_r, tap, zero)
                r0 = (i * KW + j) * Cin
                lhs[r0:r0 + Cin, :] = tap

        s_acc = jnp.zeros((CB, TP), jnp.float32)
        ss_acc = jnp.zeros((CB, TP), jnp.float32)
        pvalid = jax.lax.broadcasted_iota(jnp.int32, (CB, TP), 1)
        for c in range(n_chunks):
            y = jnp.dot(w_mat, lhs[:, c * TP:(c + 1) * TP],
                        preferred_element_type=jnp.float32)
            y_scr[:, c * TP:(c + 1) * TP] = y
            if (c + 1) * TP > P:
                y = jnp.where(pvalid < (P - c * TP), y, 0.0)
            s_acc = s_acc + y
            ss_acc = ss_acc + y * y

        inv_p = 1.0 / float(P)
        s = jnp.sum(s_acc, axis=1, keepdims=True)
        ss = jnp.sum(ss_acc, axis=1, keepdims=True)
        mean = s * inv_p
        var = jnp.maximum(ss * inv_p - mean * mean, 0.0)
        scale = jax.lax.rsqrt(var + EPS) * g_ref[...]
        shift = bt_ref[...] - mean * scale

        o_ref[0] = jnp.maximum(y_scr[:, :P] * scale + shift,
                               0.0).astype(jnp.bfloat16)

    return _body


@functools.partial(jax.jit, static_argnames=("stride", "padding"))
def _fused_conv_in_relu(x_nchw, w_oihw, gamma, beta, *, stride=1, padding=0):
    N, Cin, H, W = x_nchw.shape
    Cout, Cin_w, KH, KW = w_oihw.shape
    assert Cin == Cin_w and stride == 1
    assert padding == KH // 2 == KW // 2, "same-size conv expected"
    assert PAD >= padding * W + padding

    Ho, Wo = H, W
    P = Ho * Wo
    K = KH * KW * Cin
    CB = 128
    Cp = _round_up(Cout, CB)
    assert Cp == CB, "single 128-channel block expected"
    Pr = _round_up(P, TP)

    x_flat = x_nchw.reshape(N, Cin, P).astype(jnp.bfloat16)
    w = jnp.transpose(w_oihw, (0, 2, 3, 1)).reshape(Cout, K)
    w = jnp.pad(w, ((0, Cp - Cout), (0, 0))).astype(jnp.bfloat16)
    gp = jnp.pad(gamma, (0, Cp - Cout)).reshape(Cp, 1)
    btp = jnp.pad(beta, (0, Cp - Cout)).reshape(Cp, 1)

    body = _make_fused_kernel(KH, KW, Ho, Wo, Cin, CB)

    out = pl.pallas_call(
        body,
        out_shape=jax.ShapeDtypeStruct((N, Cp, P), jnp.bfloat16),
        grid=(N,),
        in_specs=[
            pl.BlockSpec((1, Cin, P), lambda n: (n, 0, 0)),
            pl.BlockSpec((CB, K), lambda n: (0, 0)),
            pl.BlockSpec((CB, 1), lambda n: (0, 0)),
            pl.BlockSpec((CB, 1), lambda n: (0, 0)),
        ],
        out_specs=pl.BlockSpec((1, CB, P), lambda n: (n, 0, 0)),
        scratch_shapes=[
            pltpu.VMEM((Cin, PAD + Pr + PAD), jnp.bfloat16),
            pltpu.VMEM((K, Pr), jnp.bfloat16),
            pltpu.VMEM((CB, Pr), jnp.float32),
        ],
        compiler_params=pltpu.CompilerParams(
            dimension_semantics=("parallel",)),
    )(x_flat, w, gp, btp)

    return out[:, :Cout, :].reshape(N, Cout, Ho, Wo).astype(jnp.float32)


def kernel(x, w, b, gamma, beta):
    del b
    return _fused_conv_in_relu(x, w, gamma, beta, stride=1, padding=1)
